# baseline (speedup 1.0000x reference)
import functools

import jax
import jax.numpy as jnp
from jax import lax
from jax.experimental import pallas as pl
from jax.experimental.pallas import tpu as pltpu
from jax.experimental.pallas import tpu_sc as plsc

M, K, D, N = 10000, 32, 128, 10000
L = 16
CH = D // L
NC, NS = 2, 16
NW = NC * NS
R = 4
CHUNK_IDX = R * K
NCHUNK = M // R
GPW = 80
PADC = NW * GPW
MPAD = PADC * R

_mesh = plsc.VectorSubcoreMesh(
    core_axis_name="c", subcore_axis_name="s", num_cores=NC, num_subcores=NS
)


@functools.partial(
    pl.kernel,
    out_type=jax.ShapeDtypeStruct((MPAD, D), jnp.float32),
    mesh=_mesh,
    scratch_types=[
        pltpu.VMEM((GPW, CHUNK_IDX), jnp.int32),
        pltpu.VMEM((2, CHUNK_IDX, D), jnp.float32),
        pltpu.VMEM((2, R, D), jnp.float32),
        pltpu.SemaphoreType.DMA,
        pltpu.SemaphoreType.DMA,
        pltpu.SemaphoreType.DMA,
        pltpu.SemaphoreType.DMA,
    ],
)
def _maxpool_sc(feats_hbm, idx_hbm, out_hbm, idx_all, rows_v, out_v,
                sem_g0, sem_g1, sem_o0, sem_o1):
    wid = lax.axis_index("s") * NC + lax.axis_index("c")
    base = GPW * wid

    pltpu.sync_copy(idx_hbm.at[pl.ds(base, GPW)], idx_all)

    gsems = (sem_g0, sem_g1)
    osems = (sem_o0, sem_o1)

    def issue_gather(g, b):
        pltpu.async_copy(
            feats_hbm.at[idx_all.at[g]], rows_v.at[b], gsems[b]
        )

    def wait_gather(b):
        pltpu.make_async_copy(
            feats_hbm.at[idx_all.at[0]], rows_v.at[b], gsems[b]
        ).wait()

    def wait_out(b):
        pltpu.make_async_copy(
            out_v.at[b], out_hbm.at[pl.ds(0, R)], osems[b]
        ).wait()

    issue_gather(0, 0)

    @pl.loop(0, GPW, step=2)
    def _chunk_loop(g):
        for b in range(2):
            gg = g + b

            @pl.when(gg + 1 < GPW)
            def _():
                issue_gather(gg + 1, 1 - b)

            wait_gather(b)

            @pl.when(gg >= 2)
            def _():
                wait_out(b)

            for r in range(R):
                init = tuple(
                    rows_v[b, r * K, pl.ds(c * L, L)] for c in range(CH)
                )

                def k_body(k, accs, _r=r):
                    return tuple(
                        jnp.maximum(
                            accs[c], rows_v[b, _r * K + k, pl.ds(c * L, L)]
                        )
                        for c in range(CH)
                    )

                accs = lax.fori_loop(1, K, k_body, init, unroll=4)
                for c in range(CH):
                    out_v[b, r, pl.ds(c * L, L)] = accs[c]
            pltpu.async_copy(
                out_v.at[b], out_hbm.at[pl.ds((base + gg) * R, R)], osems[b]
            )

    wait_out(0)
    wait_out(1)


def kernel(s_feats, neighbor_indices):
    idx = neighbor_indices.astype(jnp.int32).reshape(NCHUNK, CHUNK_IDX)
    idx = jnp.pad(idx, ((0, PADC - NCHUNK), (0, 0)))
    return _maxpool_sc(s_feats, idx)[:M]
